# baseline (speedup 1.0000x reference)
import functools

import jax
import jax.numpy as jnp
import numpy as np
from jax import lax
from jax.experimental import pallas as pl
from jax.experimental.pallas import tpu as pltpu
from jax.experimental.pallas import tpu_sc as plsc

P, B, K, D = 16, 128, 128, 64
N = P * B
NC, NS = 2, 16
NW = NC * NS
RPW = N // NW
LANES = 16


def _sc_gather_body(pi_hbm, mus_hbm, sig_hbm, z_hbm,
                    pis_out, mu_out, sig_out,
                    z_v, idx_v, idx2_v, rows_pi, rows_mu, rows_sig,
                    sem_pi, sem_mu, sem_sig):
    wid = lax.axis_index("s") * NC + lax.axis_index("c")
    base = wid * RPW
    pltpu.sync_copy(z_hbm.at[pl.ds(base, RPW)], z_v)
    for j in range(RPW // LANES):
        zv = z_v[pl.ds(j * LANES, LANES)]
        i16 = lax.iota(jnp.int32, LANES) + (base + j * LANES)
        idx_v[pl.ds(j * LANES, LANES)] = i16 * K + zv
        idx2_v[pl.ds(j * LANES, LANES)] = i16 * (K // 2) + lax.shift_right_logical(zv, 1)
    c_pi = pltpu.async_copy(pi_hbm.at[idx_v], rows_pi, sem_pi)
    c_mu = pltpu.async_copy(mus_hbm.at[idx2_v], rows_mu, sem_mu)
    c_sg = pltpu.async_copy(sig_hbm.at[idx2_v], rows_sig, sem_sig)
    c_pi.wait()
    c_mu.wait()
    c_sg.wait()
    pltpu.sync_copy(rows_pi, pis_out.at[pl.ds(base, RPW)])
    pltpu.sync_copy(rows_mu, mu_out.at[pl.ds(base, RPW)])
    pltpu.sync_copy(rows_sig, sig_out.at[pl.ds(base, RPW)])


_sc_gather = pl.kernel(
    _sc_gather_body,
    out_type=(
        jax.ShapeDtypeStruct((N, K), jnp.float32),
        jax.ShapeDtypeStruct((N, 2 * D), jnp.float32),
        jax.ShapeDtypeStruct((N, 2 * D), jnp.float32),
    ),
    mesh=plsc.VectorSubcoreMesh(core_axis_name="c", subcore_axis_name="s"),
    scratch_types=(
        pltpu.VMEM((RPW,), jnp.int32),
        pltpu.VMEM((RPW,), jnp.int32),
        pltpu.VMEM((RPW,), jnp.int32),
        pltpu.VMEM((RPW, K), jnp.float32),
        pltpu.VMEM((RPW, 2 * D), jnp.float32),
        pltpu.VMEM((RPW, 2 * D), jnp.float32),
        pltpu.SemaphoreType.DMA,
        pltpu.SemaphoreType.DMA,
        pltpu.SemaphoreType.DMA,
    ),
)


def _tc_math_body(pis_ref, g_ref, mupair_ref, sigpair_ref, data_ref, z_ref,
                  zs_ref, lj_ref):
    lp = jnp.log(pis_ref[:])
    y = lp + g_ref[:]
    m = jnp.max(y, axis=-1, keepdims=True)
    kio = lax.broadcasted_iota(jnp.int32, (P, B, K), 2)
    zs = jnp.min(jnp.where(y == m, kio, K), axis=-1)
    zs_ref[:] = zs
    sel = kio == zs[:, :, None]
    log_pz = jnp.sum(jnp.where(sel, lp, 0.0), axis=-1)
    odd = (z_ref[:] & 1)[:, :, None] == 1
    mu = jnp.where(odd, mupair_ref[:, :, D:], mupair_ref[:, :, :D])
    sig = jnp.where(odd, sigpair_ref[:, :, D:], sigpair_ref[:, :, :D])
    t = (data_ref[:] - mu) / sig
    log_px = jnp.sum(
        -0.5 * t * t - jnp.log(sig) - np.float32(0.5 * np.log(2.0 * np.pi)),
        axis=-1,
    )
    lj_ref[:] = log_pz + log_px


_tc_math = pl.pallas_call(
    _tc_math_body,
    out_shape=(
        jax.ShapeDtypeStruct((P, B), jnp.int32),
        jax.ShapeDtypeStruct((P, B), jnp.float32),
    ),
)


@jax.jit
def kernel(mus, sigmas, pi, z, data):
    g = jax.random.gumbel(jax.random.key(42), (P, B, K), jnp.float32)
    zf = z.reshape(N).astype(jnp.int32)
    pis_g, mupair_g, sigpair_g = _sc_gather(
        pi.reshape(N * K, K),
        mus.reshape(N * K // 2, 2 * D),
        sigmas.reshape(N * K // 2, 2 * D),
        zf,
    )
    zs, log_joint = _tc_math(
        pis_g.reshape(P, B, K), g,
        mupair_g.reshape(P, B, 2 * D), sigpair_g.reshape(P, B, 2 * D),
        data, zf.reshape(P, B))
    return (mus, sigmas, pi, zs, log_joint)


# speedup vs baseline: 2.3613x; 2.3613x over previous
import functools

import jax
import jax.numpy as jnp
import numpy as np
from jax import lax
from jax.experimental import pallas as pl
from jax.experimental.pallas import tpu as pltpu
from jax.experimental.pallas import tpu_sc as plsc

P, B, K, D = 16, 128, 128, 64
N = P * B
NC, NS = 2, 16
NW = NC * NS
RPW = N // NW
LANES = 16

GRID = 32
PPB = N // GRID
RB = PPB * D


def _sc_gather_body(pi_hbm, z_hbm, pis_out, z_v, idx_v, rows_pi, sem_pi):
    wid = lax.axis_index("s") * NC + lax.axis_index("c")
    base = wid * RPW
    pltpu.sync_copy(z_hbm.at[pl.ds(base, RPW)], z_v)
    for j in range(RPW // LANES):
        zv = z_v[pl.ds(j * LANES, LANES)]
        i16 = lax.iota(jnp.int32, LANES) + (base + j * LANES)
        idx_v[pl.ds(j * LANES, LANES)] = i16 * K + zv
    pltpu.async_copy(pi_hbm.at[idx_v], rows_pi, sem_pi).wait()
    pltpu.sync_copy(rows_pi, pis_out.at[pl.ds(base, RPW)])


_sc_gather = pl.kernel(
    _sc_gather_body,
    out_type=jax.ShapeDtypeStruct((N, K), jnp.float32),
    mesh=plsc.VectorSubcoreMesh(core_axis_name="c", subcore_axis_name="s"),
    scratch_types=(
        pltpu.VMEM((RPW,), jnp.int32),
        pltpu.VMEM((RPW,), jnp.int32),
        pltpu.VMEM((RPW, K), jnp.float32),
        pltpu.SemaphoreType.DMA,
    ),
)

_HALF_LOG_2PI = np.float32(0.5 * np.log(2.0 * np.pi))


def _tc_main_body(musT_ref, sigT_ref, pis_ref, g_ref, data_ref, oh_ref,
                  mus_out_ref, sig_out_ref, zs_ref, lj_ref):
    mus_blk = musT_ref[:]
    sig_blk = sigT_ref[:]
    mus_out_ref[:] = mus_blk
    sig_out_ref[:] = sig_blk

    oh3 = oh_ref[:].reshape(PPB, 1, K)
    mu = jnp.sum(mus_blk.reshape(PPB, D, K) * oh3, axis=2)
    sig = jnp.sum(sig_blk.reshape(PPB, D, K) * oh3, axis=2)

    lp = jnp.log(pis_ref[:])
    y = lp + g_ref[:]
    m = jnp.max(y, axis=1, keepdims=True)
    kio = lax.broadcasted_iota(jnp.int32, (PPB, K), 1)
    zs = jnp.min(jnp.where(y == m, kio, K), axis=1)
    zs_ref[0, 0, :] = zs
    log_pz = jnp.sum(jnp.where(kio == zs[:, None], lp, 0.0), axis=1)

    t = (data_ref[:] - mu) / sig
    log_px = jnp.sum(-0.5 * t * t - jnp.log(sig) - _HALF_LOG_2PI, axis=1)
    lj_ref[0, 0, :] = log_pz + log_px


_tc_main = pl.pallas_call(
    _tc_main_body,
    grid=(GRID,),
    in_specs=[
        pl.BlockSpec((RB, K), lambda i: (i, 0)),
        pl.BlockSpec((RB, K), lambda i: (i, 0)),
        pl.BlockSpec((PPB, K), lambda i: (i, 0)),
        pl.BlockSpec((PPB, K), lambda i: (i, 0)),
        pl.BlockSpec((PPB, D), lambda i: (i, 0)),
        pl.BlockSpec((PPB, K), lambda i: (i, 0)),
    ],
    out_specs=[
        pl.BlockSpec((RB, K), lambda i: (i, 0)),
        pl.BlockSpec((RB, K), lambda i: (i, 0)),
        pl.BlockSpec((1, 1, PPB), lambda i: (i, 0, 0)),
        pl.BlockSpec((1, 1, PPB), lambda i: (i, 0, 0)),
    ],
    out_shape=(
        jax.ShapeDtypeStruct((N * D, K), jnp.float32),
        jax.ShapeDtypeStruct((N * D, K), jnp.float32),
        jax.ShapeDtypeStruct((GRID, 1, PPB), jnp.int32),
        jax.ShapeDtypeStruct((GRID, 1, PPB), jnp.float32),
    ),
)


@jax.jit
def kernel(mus, sigmas, pi, z, data):
    g = jax.random.gumbel(jax.random.key(42), (P, B, K), jnp.float32)
    zf = z.reshape(N).astype(jnp.int32)
    pis_g = _sc_gather(pi.reshape(N * K, K), zf)
    onehot = (zf[:, None] == jnp.arange(K, dtype=jnp.int32)[None, :])
    musT = jnp.transpose(mus, (0, 1, 3, 2)).reshape(N * D, K)
    sigT = jnp.transpose(sigmas, (0, 1, 3, 2)).reshape(N * D, K)
    mus_o, sig_o, zs, lj = _tc_main(
        musT, sigT, pis_g, g.reshape(N, K), data.reshape(N, D),
        onehot.astype(jnp.float32))
    mus_out = jnp.transpose(mus_o.reshape(P, B, D, K), (0, 1, 3, 2))
    sig_out = jnp.transpose(sig_o.reshape(P, B, D, K), (0, 1, 3, 2))
    return (mus_out, sig_out, pi, zs.reshape(P, B), lj.reshape(P, B))
